# Initial kernel scaffold; baseline (speedup 1.0000x reference)
import jax
import jax.numpy as jnp
from jax.experimental import pallas as pl


def kernel(x, index):
    raise NotImplementedError("write your pallas kernel here")


# speedup vs baseline: 1.2680x; 1.2680x over previous
import functools

import jax
import jax.numpy as jnp
from jax import lax
from jax.experimental import pallas as pl
from jax.experimental.pallas import tpu as pltpu
from jax.experimental.pallas import tpu_sc as plsc

B, S, D = 16384, 6, 512
R = B * S
NC, NS, L = 2, 16, 16
NW = NC * NS
RPW = R // NW
CR = 96
NCHUNK = RPW // CR


TW = 48


@functools.partial(
    pl.kernel,
    out_type=jax.ShapeDtypeStruct((R, D), jnp.float32),
    mesh=plsc.VectorSubcoreMesh(core_axis_name="c", subcore_axis_name="s"),
    scratch_types=[
        pltpu.VMEM((TW,), jnp.int32),
        pltpu.VMEM((NCHUNK, CR), jnp.int32),
        pltpu.VMEM((CR, D), jnp.float32),
        pltpu.SemaphoreType.DMA,
    ],
)
def _shuffle_rows(x_hbm, tab_hbm, out_hbm, tab_v, idx_v, buf, sem):
    wid = lax.axis_index("s") * NC + lax.axis_index("c")
    wbase = wid * RPW

    pltpu.sync_copy(tab_hbm, tab_v)
    off = [tab_v[pl.ds(j * L, L)] for j in range(3)]

    def chunk_body(c, _):
        base = wbase + c * CR
        for j in range(CR // L):
            idx_v[c, pl.ds(j * L, L)] = off[j % 3] + (base + (j // 3) * TW)
        pltpu.async_copy(x_hbm.at[idx_v.at[c]], buf, sem).wait()
        pltpu.sync_copy(buf, out_hbm.at[pl.ds(base, CR)])
        return _

    lax.fori_loop(0, NCHUNK, chunk_body, 0)


def kernel(x, index):
    t = jnp.arange(TW, dtype=jnp.int32)
    tab = t - t % S + index.astype(jnp.int32)[t % S]
    out2d = _shuffle_rows(x.reshape(R, D), tab)
    return out2d.reshape(B, S, D)


# speedup vs baseline: 1.2860x; 1.0141x over previous
import functools

import jax
import jax.numpy as jnp
from jax import lax
from jax.experimental import pallas as pl
from jax.experimental.pallas import tpu as pltpu
from jax.experimental.pallas import tpu_sc as plsc

B, S, D = 16384, 6, 512
R = B * S
NC, NS, L = 2, 16, 16
NW = NC * NS
RPW = R // NW
CR = 96
NCHUNK = RPW // CR


TW = 48


@functools.partial(
    pl.kernel,
    out_type=jax.ShapeDtypeStruct((R, D), jnp.float32),
    mesh=plsc.VectorSubcoreMesh(core_axis_name="c", subcore_axis_name="s"),
    scratch_types=[
        pltpu.VMEM((TW,), jnp.int32),
        pltpu.VMEM((NCHUNK, CR), jnp.int32),
        pltpu.VMEM((CR, D), jnp.float32),
        pltpu.VMEM((CR, D), jnp.float32),
        pltpu.SemaphoreType.DMA,
        pltpu.SemaphoreType.DMA,
        pltpu.SemaphoreType.DMA,
        pltpu.SemaphoreType.DMA,
    ],
)
def _shuffle_rows(x_hbm, tab_hbm, out_hbm, tab_v, idx_v,
                  buf0, buf1, g0, g1, w0, w1):
    wid = lax.axis_index("s") * NC + lax.axis_index("c")
    wbase = wid * RPW

    pltpu.sync_copy(tab_hbm, tab_v)
    off = [tab_v[pl.ds(j * L, L)] for j in range(3)]

    def idx_body(c, _):
        base = wbase + c * CR
        for j in range(CR // L):
            idx_v[c, pl.ds(j * L, L)] = off[j % 3] + (base + (j // 3) * TW)
        return _

    lax.fori_loop(0, NCHUNK, idx_body, 0)

    buf = (buf0, buf1)
    gsem = (g0, g1)
    wsem = (w0, w1)

    def gather(c):
        return pltpu.async_copy(x_hbm.at[idx_v.at[c]], buf[c % 2], gsem[c % 2])

    def write(c):
        return pltpu.async_copy(
            buf[c % 2], out_hbm.at[pl.ds(wbase + c * CR, CR)], wsem[c % 2])

    gh = [None, None]
    wh = [None, None]
    gh[0] = gather(0)
    for c in range(NCHUNK):
        b = c % 2
        gh[b].wait()
        wh[b] = write(c)
        if c + 1 < NCHUNK:
            nb = (c + 1) % 2
            if wh[nb] is not None:
                wh[nb].wait()
            gh[nb] = gather(c + 1)
    wh[0].wait()
    wh[1].wait()


def kernel(x, index):
    t = jnp.arange(TW, dtype=jnp.int32)
    tab = t - t % S + index.astype(jnp.int32)[t % S]
    out2d = _shuffle_rows(x.reshape(R, D), tab)
    return out2d.reshape(B, S, D)
